# Initial kernel scaffold; baseline (speedup 1.0000x reference)
import jax
import jax.numpy as jnp
from jax.experimental import pallas as pl


def kernel(in_feat, edge_index, e_weight, target_node, W1, b1, W2, b2, W3, b3):
    raise NotImplementedError("write your pallas kernel here")


# speedup vs baseline: 4.9187x; 4.9187x over previous
import functools

import jax
import jax.numpy as jnp
from jax import lax
from jax.experimental import pallas as pl
from jax.experimental.pallas import tpu as pltpu
from jax.experimental.pallas import tpu_sc as plsc

NN = 10000
EE = 320000
DD = 128
H3 = 16
NW = 32
EPT = EE // NW
K = 80
NCH = EPT // K
TP = 1024

_f32 = jnp.float32
_i32 = jnp.int32


_SC_PARAMS = pltpu.CompilerParams(needs_layout_passes=False, use_tc_tiling_on_sc=False)


def _mesh():
    return plsc.VectorSubcoreMesh(core_axis_name="c", subcore_axis_name="s")


def _stripe(sid):
    base = sid * 640
    nch = jnp.where(sid == 15, 5, 8)
    return base, nch


def _off(base, k):
    return pl.multiple_of(base + k * 80, 8)


@functools.partial(
    pl.kernel,
    out_type=jax.ShapeDtypeStruct((2, NN, 16), _f32),
    mesh=_mesh(),
    compiler_params=_SC_PARAMS,
    scratch_types=[
        pltpu.VMEM((NCH, K), _i32),
        pltpu.VMEM((NCH, K), _i32),
        pltpu.VMEM((K, 16), _f32),
        pltpu.VMEM((K, 16), _f32),
        pltpu.VMEM((80, 16), _f32),
        pltpu.VMEM_SHARED((NN, 16), _f32),
    ],
)
def _hist(src_hbm, dst_hbm, out_hbm, src_b, dst_b, e0_b, e1_b, zb, acc):
    c = lax.axis_index("c")
    sid = lax.axis_index("s")
    g = c * 16 + sid
    pltpu.sync_copy(src_hbm.at[g], src_b)
    pltpu.sync_copy(dst_hbm.at[g], dst_b)
    iot = lax.iota(_i32, 16)
    v0 = jnp.where(iot == 0, 1.0, 0.0).astype(_f32)
    v1 = jnp.where(iot == 1, 1.0, 0.0).astype(_f32)
    zv = jnp.zeros((16,), _f32)

    def initrow(r, _):
        e0_b[r, :] = v0
        e1_b[r, :] = v1
        return 0

    lax.fori_loop(0, K, initrow, 0)

    def zrow(r, _):
        zb[r, :] = zv
        return 0

    lax.fori_loop(0, 80, zrow, 0)
    base, nst = _stripe(sid)

    def zst(k, _):
        pltpu.sync_copy(zb, acc.at[pl.ds(_off(base, k), 80)])
        return 0

    lax.fori_loop(0, nst, zst, 0)
    plsc.subcore_barrier()

    def chunk(j, _):
        pltpu.sync_copy(e0_b, acc.at[src_b.at[j]], add=True)
        pltpu.sync_copy(e1_b, acc.at[dst_b.at[j]], add=True)
        return 0

    lax.fori_loop(0, NCH, chunk, 0)
    plsc.subcore_barrier()

    def ost(k, _):
        sl = pl.ds(_off(base, k), 80)
        pltpu.sync_copy(acc.at[sl], out_hbm.at[c, sl])
        return 0

    lax.fori_loop(0, nst, ost, 0)


def _make_agg(h, split_cols):
    nch = 2 * NCH if split_cols else NCH

    @functools.partial(
        pl.kernel,
        out_type=jax.ShapeDtypeStruct((2, NN, h), _f32),
        mesh=_mesh(),
        compiler_params=_SC_PARAMS,
        scratch_types=[
            pltpu.VMEM((nch, K), _i32),
            pltpu.VMEM((nch, K), _i32),
            pltpu.VMEM((nch, K), _f32),
            pltpu.VMEM((NN * 2,), _f32),
            pltpu.VMEM((K, h), _f32),
            pltpu.VMEM_SHARED((NN, h), _f32),
            pltpu.SemaphoreType.DMA,
        ],
    )
    def agg(z_hbm, src_hbm, dst_hbm, w_hbm, norms_hbm, out_hbm,
            src_b, dst_b, w_b, norm_b, rows, acc, sem):
        c = lax.axis_index("c")
        sid = lax.axis_index("s")
        g = sid if split_cols else c * 16 + sid
        roff = c * NN if split_cols else 0
        pltpu.sync_copy(src_hbm.at[g], src_b)
        pltpu.sync_copy(dst_hbm.at[g], dst_b)
        pltpu.sync_copy(w_hbm.at[g], w_b)
        pltpu.sync_copy(norms_hbm, norm_b)

        def fold(t, _):
            j = t // (K // 16)
            u = t % (K // 16)
            sl = pl.ds(u * 16, 16)
            s16 = src_b[j, sl]
            on = plsc.load_gather(norm_b, [s16 * 2])
            w_b[j, sl] = w_b[j, sl] * on
            src_b[j, sl] = s16 + roff
            return 0

        lax.fori_loop(0, nch * (K // 16), fold, 0)

        zv = jnp.zeros((16,), _f32)

        def zrow(r, _):
            for k in range(h // 16):
                rows[r, pl.ds(k * 16, 16)] = zv
            return 0

        lax.fori_loop(0, K, zrow, 0)
        base, nst = _stripe(sid)

        def zst(k, _):
            pltpu.sync_copy(rows, acc.at[pl.ds(_off(base, k), 80)])
            return 0

        lax.fori_loop(0, nst, zst, 0)
        plsc.subcore_barrier()

        def chunk(j, _):
            pltpu.async_copy(z_hbm.at[src_b.at[j]], rows, sem).wait()

            def grp(u, _):
                s16 = w_b[j, pl.ds(u * 16, 16)]
                for t in range(16):
                    e = u * 16 + t
                    s = s16[t]
                    for k in range(h // 16):
                        sl = pl.ds(k * 16, 16)
                        rows[e, sl] = rows[e, sl] * s
                return 0

            lax.fori_loop(0, K // 16, grp, 0)
            pltpu.sync_copy(rows, acc.at[dst_b.at[j]], add=True)
            return 0

        lax.fori_loop(0, nch, chunk, 0)
        plsc.subcore_barrier()

        def ost(k, _):
            sl = pl.ds(_off(base, k), 80)
            pltpu.sync_copy(acc.at[sl], out_hbm.at[c, sl])
            return 0

        lax.fori_loop(0, nst, ost, 0)

    return agg


_agg64 = _make_agg(DD // 2, True)
_agg16 = _make_agg(H3, False)


@functools.partial(
    pl.kernel,
    out_type=jax.ShapeDtypeStruct((TP, H3), _f32),
    mesh=_mesh(),
    compiler_params=_SC_PARAMS,
    scratch_types=[
        pltpu.VMEM((TP // NW,), _i32),
        pltpu.VMEM((TP // NW,), _i32),
        pltpu.VMEM((TP // NW, H3), _f32),
        pltpu.VMEM((TP // NW, H3), _f32),
        pltpu.VMEM((TP // NW, H3), _f32),
        pltpu.VMEM((NN * 2,), _f32),
        pltpu.VMEM((H3,), _f32),
        pltpu.SemaphoreType.DMA,
    ],
)
def _final(pcat_hbm, norms_hbm, b3_hbm, tgt_hbm, out_hbm,
           ti, ti2, ra, rb, ro, nb, bb, sem):
    c = lax.axis_index("c")
    sid = lax.axis_index("s")
    w = c * 16 + sid
    bpw = TP // NW
    pltpu.sync_copy(tgt_hbm.at[pl.ds(w * bpw, bpw)], ti)
    pltpu.sync_copy(norms_hbm, nb)
    pltpu.sync_copy(b3_hbm, bb)
    for u in range(bpw // 16):
        sl = pl.ds(u * 16, 16)
        ti2[sl] = ti[sl] + NN
    pltpu.async_copy(pcat_hbm.at[ti], ra, sem).wait()
    pltpu.async_copy(pcat_hbm.at[ti2], rb, sem).wait()
    bv = bb[:]

    for u in range(bpw // 16):
        t16 = ti[pl.ds(u * 16, 16)]
        nv16 = plsc.load_gather(nb, [t16 * 2 + 1])
        for t in range(16):
            i = u * 16 + t
            nv = nv16[t]
            ro[i, :] = (ra[i, :] + rb[i, :]) * nv + bv
    pltpu.sync_copy(ro, out_hbm.at[pl.ds(w * bpw, bpw)])


_BR = 400
_NG = NN // _BR


def _tc_prep(x, w1, ph):
    def body(x_ref, w_ref, ph_ref, z_ref, n_ref):
        z = jnp.dot(x_ref[...], w_ref[...], preferred_element_type=_f32)
        z_ref[0] = z[:, :DD // 2]
        z_ref[1] = z[:, DD // 2:]
        co = ph_ref[0, :, 0:1] + ph_ref[1, :, 0:1]
        ci = ph_ref[0, :, 1:2] + ph_ref[1, :, 1:2]
        n_ref[:, 0:1] = lax.rsqrt(jnp.maximum(co, 1.0))
        n_ref[:, 1:2] = lax.rsqrt(jnp.maximum(ci, 1.0))

    return pl.pallas_call(
        body,
        grid=(_NG,),
        in_specs=[
            pl.BlockSpec((_BR, DD), lambda i: (i, 0)),
            pl.BlockSpec((DD, DD), lambda i: (0, 0)),
            pl.BlockSpec((2, _BR, 16), lambda i: (0, i, 0)),
        ],
        out_specs=[
            pl.BlockSpec((2, _BR, DD // 2), lambda i: (0, i, 0)),
            pl.BlockSpec((_BR, 2), lambda i: (i, 0)),
        ],
        out_shape=[
            jax.ShapeDtypeStruct((2, NN, DD // 2), _f32),
            jax.ShapeDtypeStruct((NN, 2), _f32),
        ],
    )(x, w1, ph)


def _tc_layer(p, norms, b, w, h_out):
    def body(p_ref, n_ref, b_ref, w_ref, z_ref):
        x = jnp.concatenate([p_ref[0], p_ref[1]], axis=1)
        x = x * n_ref[:, 1:2] + b_ref[...]
        x = jnp.maximum(x, 0.0)
        z = jnp.dot(x, w_ref[...], preferred_element_type=_f32)
        if h_out == DD:
            z_ref[0] = z[:, :DD // 2]
            z_ref[1] = z[:, DD // 2:]
        else:
            z_ref[...] = z

    if h_out == DD:
        out_spec = pl.BlockSpec((2, _BR, DD // 2), lambda i: (0, i, 0))
        out_shape = jax.ShapeDtypeStruct((2, NN, DD // 2), _f32)
    else:
        out_spec = pl.BlockSpec((_BR, h_out), lambda i: (i, 0))
        out_shape = jax.ShapeDtypeStruct((NN, h_out), _f32)

    return pl.pallas_call(
        body,
        grid=(_NG,),
        in_specs=[
            pl.BlockSpec((2, _BR, DD // 2), lambda i: (0, i, 0)),
            pl.BlockSpec((_BR, 2), lambda i: (i, 0)),
            pl.BlockSpec((1, DD), lambda i: (0, 0)),
            pl.BlockSpec((DD, h_out), lambda i: (0, 0)),
        ],
        out_specs=out_spec,
        out_shape=out_shape,
    )(p, norms, b, w)


def kernel(in_feat, edge_index, e_weight, target_node,
           W1, b1, W2, b2, W3, b3):
    src32 = edge_index[0].reshape(NW, NCH, K)
    dst32 = edge_index[1].reshape(NW, NCH, K)
    wr32 = e_weight.reshape(NW, NCH, K)
    src16 = edge_index[0].reshape(16, 2 * NCH, K)
    dst16 = edge_index[1].reshape(16, 2 * NCH, K)
    wr16 = e_weight.reshape(16, 2 * NCH, K)

    ph = _hist(src32, dst32)
    z1, norms = _tc_prep(in_feat, W1, ph)
    nflat = norms.reshape(-1)
    p1 = _agg64(z1.reshape(2 * NN, DD // 2), src16, dst16, wr16, nflat)
    z2 = _tc_layer(p1, norms, b1.reshape(1, DD), W2, DD)
    p2 = _agg64(z2.reshape(2 * NN, DD // 2), src16, dst16, wr16, nflat)
    w3p = jnp.pad(W3, ((0, 0), (0, H3 - 4)))
    b3p = jnp.pad(b3, (0, H3 - 4))
    z3 = _tc_layer(p2, norms, b2.reshape(1, DD), w3p, H3)
    p3 = _agg16(z3, src32, dst32, wr32, nflat)
    tgt = jnp.concatenate(
        [target_node, jnp.zeros((TP - 1000,), _i32)])
    outp = _final(p3.reshape(2 * NN, H3), nflat, b3p, tgt)
    return outp[:1000, :4]


# speedup vs baseline: 5.7902x; 1.1772x over previous
import functools

import jax
import jax.numpy as jnp
from jax import lax
from jax.experimental import pallas as pl
from jax.experimental.pallas import tpu as pltpu
from jax.experimental.pallas import tpu_sc as plsc

NN = 10000
EE = 320000
DD = 128
H3 = 16
NW = 32
EPT = EE // NW
K = 80
NCH = EPT // K
TP = 1024

_f32 = jnp.float32
_i32 = jnp.int32


_SC_PARAMS = pltpu.CompilerParams(needs_layout_passes=False, use_tc_tiling_on_sc=False)


def _mesh():
    return plsc.VectorSubcoreMesh(core_axis_name="c", subcore_axis_name="s")


def _stripe(sid):
    base = sid * 640
    nch = jnp.where(sid == 15, 5, 8)
    return base, nch


def _off(base, k):
    return pl.multiple_of(base + k * 80, 8)


@functools.partial(
    pl.kernel,
    out_type=jax.ShapeDtypeStruct((2, NN, 16), _f32),
    mesh=_mesh(),
    compiler_params=_SC_PARAMS,
    scratch_types=[
        pltpu.VMEM((NCH, K), _i32),
        pltpu.VMEM((NCH, K), _i32),
        pltpu.VMEM((K, 16), _f32),
        pltpu.VMEM((K, 16), _f32),
        pltpu.VMEM((80, 16), _f32),
        pltpu.VMEM_SHARED((NN, 16), _f32),
    ],
)
def _hist(src_hbm, dst_hbm, out_hbm, src_b, dst_b, e0_b, e1_b, zb, acc):
    c = lax.axis_index("c")
    sid = lax.axis_index("s")
    g = c * 16 + sid
    pltpu.sync_copy(src_hbm.at[g], src_b)
    pltpu.sync_copy(dst_hbm.at[g], dst_b)
    iot = lax.iota(_i32, 16)
    v0 = jnp.where(iot == 0, 1.0, 0.0).astype(_f32)
    v1 = jnp.where(iot == 1, 1.0, 0.0).astype(_f32)
    zv = jnp.zeros((16,), _f32)

    def initrow(r, _):
        e0_b[r, :] = v0
        e1_b[r, :] = v1
        return 0

    lax.fori_loop(0, K, initrow, 0)

    def zrow(r, _):
        zb[r, :] = zv
        return 0

    lax.fori_loop(0, 80, zrow, 0)
    base, nst = _stripe(sid)

    def zst(k, _):
        pltpu.sync_copy(zb, acc.at[pl.ds(_off(base, k), 80)])
        return 0

    lax.fori_loop(0, nst, zst, 0)
    plsc.subcore_barrier()

    def chunk(j, _):
        pltpu.sync_copy(e0_b, acc.at[src_b.at[j]], add=True)
        pltpu.sync_copy(e1_b, acc.at[dst_b.at[j]], add=True)
        return 0

    lax.fori_loop(0, NCH, chunk, 0)
    plsc.subcore_barrier()

    def ost(k, _):
        sl = pl.ds(_off(base, k), 80)
        pltpu.sync_copy(acc.at[sl], out_hbm.at[c, sl])
        return 0

    lax.fori_loop(0, nst, ost, 0)


def _make_agg(h, split_cols):
    nch = 2 * NCH if split_cols else NCH

    @functools.partial(
        pl.kernel,
        out_type=jax.ShapeDtypeStruct((2, NN, h), _f32),
        mesh=_mesh(),
        compiler_params=_SC_PARAMS,
        scratch_types=[
            pltpu.VMEM((nch, K), _i32),
            pltpu.VMEM((nch, K), _i32),
            pltpu.VMEM((nch, K), _f32),
            pltpu.VMEM((K, h), _f32),
            pltpu.VMEM((K, h), _f32),
            pltpu.VMEM_SHARED((NN, h), _f32),
            pltpu.SemaphoreType.DMA,
            pltpu.SemaphoreType.DMA,
            pltpu.SemaphoreType.DMA,
            pltpu.SemaphoreType.DMA,
        ],
    )
    def agg(z_hbm, src_hbm, dst_hbm, w_hbm, out_hbm,
            src_b, dst_b, w_b, rows0, rows1, acc, gs0, gs1, ss0, ss1):
        c = lax.axis_index("c")
        sid = lax.axis_index("s")
        g = sid if split_cols else c * 16 + sid
        pltpu.sync_copy(src_hbm.at[g], src_b)
        pltpu.sync_copy(dst_hbm.at[g], dst_b)
        pltpu.sync_copy(w_hbm.at[g], w_b)
        rows = (rows0, rows1)
        gs = (gs0, gs1)
        ss = (ss0, ss1)

        if split_cols:
            roff = c * NN

            @pl.when(roff > 0)
            def _():
                def fold(t, _):
                    j = t // (K // 16)
                    u = t % (K // 16)
                    sl = pl.ds(u * 16, 16)
                    src_b[j, sl] = src_b[j, sl] + roff
                    return 0

                lax.fori_loop(0, nch * (K // 16), fold, 0)

        zv = jnp.zeros((16,), _f32)

        def zrow(r, _):
            for k in range(h // 16):
                rows0[r, pl.ds(k * 16, 16)] = zv
            return 0

        lax.fori_loop(0, K, zrow, 0)
        base, nst = _stripe(sid)

        def zst(k, _):
            pltpu.sync_copy(rows0, acc.at[pl.ds(_off(base, k), 80)])
            return 0

        lax.fori_loop(0, nst, zst, 0)
        plsc.subcore_barrier()

        def gstart(j, b):
            pltpu.async_copy(z_hbm.at[src_b.at[j]], rows[b], gs[b])

        def gwait(j, b):
            pltpu.make_async_copy(z_hbm.at[src_b.at[j]], rows[b], gs[b]).wait()

        def sstart(j, b):
            pltpu.async_copy(rows[b], acc.at[dst_b.at[j]], ss[b], add=True)

        def swait(j, b):
            pltpu.make_async_copy(rows[b], acc.at[dst_b.at[j]], ss[b]).wait()

        def step(j, b):
            o = 1 - b
            gwait(j, b)

            def grp(u, _):
                s16 = w_b[j, pl.ds(u * 16, 16)]
                for t in range(16):
                    e = u * 16 + t
                    s = s16[t]
                    for k in range(h // 16):
                        sl = pl.ds(k * 16, 16)
                        rows[b][e, sl] = rows[b][e, sl] * s
                return 0

            lax.fori_loop(0, K // 16, grp, 0)
            sstart(j, b)

            @pl.when(j + 1 < nch)
            def _():
                @pl.when(j >= 1)
                def _():
                    swait(j - 1, o)

                gstart(j + 1, o)

            @pl.when(j == nch - 1)
            def _():
                swait(j - 1, o)
                swait(j, b)

        gstart(jnp.int32(0), 0)

        def pair(i, _):
            step(2 * i, 0)
            step(2 * i + 1, 1)
            return 0

        lax.fori_loop(0, nch // 2, pair, 0)
        if nch % 2:
            step(jnp.int32(nch - 1), 0)
        plsc.subcore_barrier()

        def ost(k, _):
            sl = pl.ds(_off(base, k), 80)
            pltpu.sync_copy(acc.at[sl], out_hbm.at[c, sl])
            return 0

        lax.fori_loop(0, nst, ost, 0)

    return agg


_agg64 = _make_agg(DD // 2, True)
_agg16 = _make_agg(H3, False)


@functools.partial(
    pl.kernel,
    out_type=jax.ShapeDtypeStruct((TP, H3), _f32),
    mesh=_mesh(),
    compiler_params=_SC_PARAMS,
    scratch_types=[
        pltpu.VMEM((TP // NW,), _i32),
        pltpu.VMEM((TP // NW,), _i32),
        pltpu.VMEM((TP // NW, H3), _f32),
        pltpu.VMEM((TP // NW, H3), _f32),
        pltpu.VMEM((TP // NW, H3), _f32),
        pltpu.VMEM((NN * 2,), _f32),
        pltpu.VMEM((H3,), _f32),
        pltpu.SemaphoreType.DMA,
    ],
)
def _final(pcat_hbm, norms_hbm, b3_hbm, tgt_hbm, out_hbm,
           ti, ti2, ra, rb, ro, nb, bb, sem):
    c = lax.axis_index("c")
    sid = lax.axis_index("s")
    w = c * 16 + sid
    bpw = TP // NW
    pltpu.sync_copy(tgt_hbm.at[pl.ds(w * bpw, bpw)], ti)
    pltpu.sync_copy(norms_hbm, nb)
    pltpu.sync_copy(b3_hbm, bb)
    for u in range(bpw // 16):
        sl = pl.ds(u * 16, 16)
        ti2[sl] = ti[sl] + NN
    pltpu.async_copy(pcat_hbm.at[ti], ra, sem).wait()
    pltpu.async_copy(pcat_hbm.at[ti2], rb, sem).wait()
    bv = bb[:]

    for u in range(bpw // 16):
        t16 = ti[pl.ds(u * 16, 16)]
        nv16 = plsc.load_gather(nb, [t16 * 2 + 1])
        for t in range(16):
            i = u * 16 + t
            nv = nv16[t]
            ro[i, :] = (ra[i, :] + rb[i, :]) * nv + bv
    pltpu.sync_copy(ro, out_hbm.at[pl.ds(w * bpw, bpw)])


_BR = 400
_NG = NN // _BR


def _tc_prep(x, w1, ph):
    def body(x_ref, w_ref, ph_ref, z_ref, n_ref):
        co = ph_ref[0, :, 0:1] + ph_ref[1, :, 0:1]
        ci = ph_ref[0, :, 1:2] + ph_ref[1, :, 1:2]
        on = lax.rsqrt(jnp.maximum(co, 1.0))
        n_ref[:, 0:1] = on
        n_ref[:, 1:2] = lax.rsqrt(jnp.maximum(ci, 1.0))
        z = jnp.dot(x_ref[...], w_ref[...], preferred_element_type=_f32)
        z = z * on
        z_ref[0] = z[:, :DD // 2]
        z_ref[1] = z[:, DD // 2:]

    return pl.pallas_call(
        body,
        grid=(_NG,),
        in_specs=[
            pl.BlockSpec((_BR, DD), lambda i: (i, 0)),
            pl.BlockSpec((DD, DD), lambda i: (0, 0)),
            pl.BlockSpec((2, _BR, 16), lambda i: (0, i, 0)),
        ],
        out_specs=[
            pl.BlockSpec((2, _BR, DD // 2), lambda i: (0, i, 0)),
            pl.BlockSpec((_BR, 2), lambda i: (i, 0)),
        ],
        out_shape=[
            jax.ShapeDtypeStruct((2, NN, DD // 2), _f32),
            jax.ShapeDtypeStruct((NN, 2), _f32),
        ],
    )(x, w1, ph)


def _tc_layer(p, norms, b, w, h_out):
    def body(p_ref, n_ref, b_ref, w_ref, z_ref):
        x = jnp.concatenate([p_ref[0], p_ref[1]], axis=1)
        x = x * n_ref[:, 1:2] + b_ref[...]
        x = jnp.maximum(x, 0.0)
        z = jnp.dot(x, w_ref[...], preferred_element_type=_f32)
        z = z * n_ref[:, 0:1]
        if h_out == DD:
            z_ref[0] = z[:, :DD // 2]
            z_ref[1] = z[:, DD // 2:]
        else:
            z_ref[...] = z

    if h_out == DD:
        out_spec = pl.BlockSpec((2, _BR, DD // 2), lambda i: (0, i, 0))
        out_shape = jax.ShapeDtypeStruct((2, NN, DD // 2), _f32)
    else:
        out_spec = pl.BlockSpec((_BR, h_out), lambda i: (i, 0))
        out_shape = jax.ShapeDtypeStruct((NN, h_out), _f32)

    return pl.pallas_call(
        body,
        grid=(_NG,),
        in_specs=[
            pl.BlockSpec((2, _BR, DD // 2), lambda i: (0, i, 0)),
            pl.BlockSpec((_BR, 2), lambda i: (i, 0)),
            pl.BlockSpec((1, DD), lambda i: (0, 0)),
            pl.BlockSpec((DD, h_out), lambda i: (0, 0)),
        ],
        out_specs=out_spec,
        out_shape=out_shape,
    )(p, norms, b, w)


def kernel(in_feat, edge_index, e_weight, target_node,
           W1, b1, W2, b2, W3, b3):
    src32 = edge_index[0].reshape(NW, NCH, K)
    dst32 = edge_index[1].reshape(NW, NCH, K)
    wr32 = e_weight.reshape(NW, NCH, K)
    src16 = edge_index[0].reshape(16, 2 * NCH, K)
    dst16 = edge_index[1].reshape(16, 2 * NCH, K)
    wr16 = e_weight.reshape(16, 2 * NCH, K)

    ph = _hist(src32, dst32)
    z1, norms = _tc_prep(in_feat, W1, ph)
    nflat = norms.reshape(-1)
    p1 = _agg64(z1.reshape(2 * NN, DD // 2), src16, dst16, wr16)
    z2 = _tc_layer(p1, norms, b1.reshape(1, DD), W2, DD)
    p2 = _agg64(z2.reshape(2 * NN, DD // 2), src16, dst16, wr16)
    w3p = jnp.pad(W3, ((0, 0), (0, H3 - 4)))
    b3p = jnp.pad(b3, (0, H3 - 4))
    z3 = _tc_layer(p2, norms, b2.reshape(1, DD), w3p, H3)
    p3 = _agg16(z3, src32, dst32, wr32)
    tgt = jnp.concatenate(
        [target_node, jnp.zeros((TP - 1000,), _i32)])
    outp = _final(p3.reshape(2 * NN, H3), nflat, b3p, tgt)
    return outp[:1000, :4]
